# Initial kernel scaffold; baseline (speedup 1.0000x reference)
import jax
import jax.numpy as jnp
from jax.experimental import pallas as pl


def kernel(atom_type, r_feat, p_feat, pos, pos_init, t, batch, edge_index, edge_type_r, edge_type_p, atom_emb_table, W_feat, W_node, edge_type_emb, W_dist, W_cat, W_msg, W_scalar):
    raise NotImplementedError("write your pallas kernel here")


# speedup vs baseline: 10.8580x; 10.8580x over previous
import functools

import jax
import jax.numpy as jnp
from jax import lax
from jax.experimental import pallas as pl
from jax.experimental.pallas import tpu as pltpu
from jax.experimental.pallas import tpu_sc as plsc

_NC = 2
_NS = 16
_LANES = 16
_CH_ROWS = 8
_NODE_BLK = 1024


def _nrsqrt(x):
    k = lax.bitcast_convert_type(x, jnp.int32)
    y = lax.bitcast_convert_type(jnp.int32(0x5F3759DF) - (k >> 1), jnp.float32)
    for _ in range(3):
        y = y * (1.5 - 0.5 * x * y * y)
    return y


def _node_body(at_ref, rf_ref, pf_ref, bt_ref, pos_ref, pi_ref, tab_ref,
               wf0_ref, wf1_ref, wf2_ref, wf3_ref, t_ref, wt1_ref, wt2_ref,
               wtt_ref, packI_ref, packJ_ref):
    f32 = jnp.float32
    B = at_ref.shape[0]
    NT = tab_ref.shape[0]
    G = t_ref.shape[0]
    aoh = (at_ref[...] == lax.broadcasted_iota(jnp.int32, (B, NT), 1)).astype(f32)
    ae = jnp.dot(aoh, tab_ref[...], preferred_element_type=f32)
    wfs = (wf0_ref, wf1_ref, wf2_ref, wf3_ref)

    def featsum(ref):
        acc = None
        for f in range(4):
            oh = (ref[:, f:f + 1] == lax.broadcasted_iota(jnp.int32, (B, 10), 1)).astype(f32)
            v = jnp.dot(oh, wfs[f][...], preferred_element_type=f32)
            acc = v if acc is None else acc + v
        return acc

    rW = featsum(rf_ref)
    pW = featsum(pf_ref)
    boh = (bt_ref[...] == lax.broadcasted_iota(jnp.int32, (B, G), 1)).astype(f32)
    tn = jnp.dot(boh, t_ref[...], preferred_element_type=f32)
    abcd = (jnp.dot(ae * rW, wt1_ref[...], preferred_element_type=f32)
            + jnp.dot(ae * pW, wt2_ref[...], preferred_element_type=f32)
            + tn * wtt_ref[...])
    packI_ref[...] = jnp.concatenate([pos_ref[...], pi_ref[...]], axis=1)
    packJ_ref[...] = jnp.concatenate(
        [pos_ref[...], pi_ref[...], abcd, jnp.zeros((B, 4), f32)], axis=1)


def _node_tables(atom_type, r_feat, p_feat, batch, pos4, pi4, table, W_feat,
                 t, Wt1, Wt2, wtt, Npad):
    nblk = Npad // _NODE_BLK
    row = lambda b: (b, 0)
    rep = lambda b: (0, 0)
    blk = lambda c: pl.BlockSpec((_NODE_BLK, c), row)
    full = lambda a: pl.BlockSpec(a.shape, rep)
    wf = [W_feat[10 * f:10 * (f + 1)] for f in range(4)]
    return pl.pallas_call(
        _node_body,
        grid=(nblk,),
        in_specs=[blk(1), blk(4), blk(4), blk(1), blk(4), blk(4),
                  full(table), full(wf[0]), full(wf[1]), full(wf[2]),
                  full(wf[3]), full(t), full(Wt1), full(Wt2), full(wtt)],
        out_specs=[blk(8), blk(32)],
        out_shape=[jax.ShapeDtypeStruct((Npad, 8), jnp.float32),
                   jax.ShapeDtypeStruct((Npad, 32), jnp.float32)],
    )(atom_type, r_feat, p_feat, batch, pos4, pi4, table, wf[0], wf[1],
      wf[2], wf[3], t, Wt1, Wt2, wtt)


def _make_edge_kernel(Npad, ER, NCH):
    NW = _NC * _NS
    rows_w = ER // NW
    mesh = plsc.VectorSubcoreMesh(core_axis_name="c", subcore_axis_name="s",
                                  num_cores=_NC, num_subcores=_NS)

    @functools.partial(
        pl.kernel,
        out_type=jax.ShapeDtypeStruct((NW, Npad, 4), jnp.float32),
        mesh=mesh,
        compiler_params=pltpu.CompilerParams(needs_layout_passes=False,
                                             use_tc_tiling_on_sc=False),
        scratch_types=[
            pltpu.VMEM((_CH_ROWS, 128), jnp.int32),
            pltpu.VMEM((_CH_ROWS, 128), jnp.int32),
            pltpu.VMEM((_CH_ROWS, 128), jnp.int32),
            pltpu.VMEM((_CH_ROWS, 128), jnp.int32),
            *[pltpu.VMEM((128,), jnp.int32) for _ in range(_CH_ROWS)],
            *[pltpu.VMEM((128,), jnp.int32) for _ in range(_CH_ROWS)],
            *[pltpu.VMEM((128, 8), jnp.float32) for _ in range(_CH_ROWS)],
            *[pltpu.VMEM((128, 32), jnp.float32) for _ in range(_CH_ROWS)],
            pltpu.VMEM((Npad, 4), jnp.float32),
            pltpu.SemaphoreType.DMA,
        ],
    )
    def edge_kernel(ei_hbm, ej_hbm, etr_hbm, etp_hbm, packI_hbm, packJ_hbm,
                    out_hbm, iv2d, jv2d, erv, epv, *rest):
        R = _CH_ROWS
        iv1d = rest[0:R]
        jv1d = rest[R:2 * R]
        bufI = rest[2 * R:3 * R]
        bufJ = rest[3 * R:4 * R]
        acc = rest[4 * R]
        semg = rest[4 * R + 1]
        cid = lax.axis_index("c")
        sid = lax.axis_index("s")
        wid = cid * _NS + sid
        lanes = lax.iota(jnp.int32, _LANES)
        cc = lambda c: jnp.full((_LANES,), c, jnp.int32)

        def zero_body(k, c):
            li = k * _LANES + lanes
            for col in range(4):
                plsc.store_scatter(acc, [li, cc(col)],
                                   jnp.zeros((_LANES,), jnp.float32))
            return c
        lax.fori_loop(0, Npad // _LANES, zero_body, 0)

        base_row = wid * rows_w

        def chunk_body(ch, carry):
            r0 = base_row + ch * _CH_ROWS
            pltpu.sync_copy(ei_hbm.at[pl.ds(r0, _CH_ROWS)], iv2d)
            pltpu.sync_copy(ej_hbm.at[pl.ds(r0, _CH_ROWS)], jv2d)
            pltpu.sync_copy(etr_hbm.at[pl.ds(r0, _CH_ROWS)], erv)
            pltpu.sync_copy(etp_hbm.at[pl.ds(r0, _CH_ROWS)], epv)
            for r in range(R):
                def ix_body(s, c2, r=r):
                    li = s * _LANES + lanes
                    rsp = jnp.full((_LANES,), r, jnp.int32)
                    plsc.store_scatter(iv1d[r], [li],
                                       plsc.load_gather(iv2d, [rsp, li]))
                    plsc.store_scatter(jv1d[r], [li],
                                       plsc.load_gather(jv2d, [rsp, li]))
                    return c2
                lax.fori_loop(0, 8, ix_body, 0)
            cps = []
            for r in range(R):
                cps.append(pltpu.async_copy(packI_hbm.at[iv1d[r]],
                                            bufI[r], semg))
                cps.append(pltpu.async_copy(packJ_hbm.at[jv1d[r]],
                                            bufJ[r], semg))
            for cp in cps:
                cp.wait()

            for r in range(R):
                def rs_body(s, c2, r=r):
                    li = s * _LANES + lanes
                    rsp = jnp.full((_LANES,), r, jnp.int32)

                    def colI(c):
                        return plsc.load_gather(bufI[r], [li, cc(c)])

                    def colJ(cv):
                        return plsc.load_gather(bufJ[r], [li, cv])

                    pix, piy, piz = colI(0), colI(1), colI(2)
                    qix, qiy, qiz = colI(4), colI(5), colI(6)
                    pjx, pjy, pjz = colJ(cc(0)), colJ(cc(1)), colJ(cc(2))
                    qjx, qjy, qjz = colJ(cc(4)), colJ(cc(5)), colJ(cc(6))
                    etr = plsc.load_gather(erv, [rsp, li])
                    etp = plsc.load_gather(epv, [rsp, li])
                    Av = colJ(8 + etr)
                    Bv = colJ(13 + etr)
                    Cv = colJ(18 + etp)
                    Dv = colJ(23 + etp)
                    ival = plsc.load_gather(iv1d[r], [li])
                    dx = pix - pjx
                    dy = piy - pjy
                    dz = piz - pjz
                    dd = dx * dx + dy * dy + dz * dz + 1e-8
                    y = _nrsqrt(dd)
                    tx = qix - qjx
                    ty = qiy - qjy
                    tz = qiz - qjz
                    ddT = tx * tx + ty * ty + tz * tz + 1e-8
                    yT = _nrsqrt(ddT)
                    coef = (dd * y) * (Av + Cv) + (ddT * yT) * (Bv + Dv)
                    sc = coef * y
                    plsc.addupdate_scatter(acc, [ival, cc(0)], sc * dx)
                    plsc.addupdate_scatter(acc, [ival, cc(1)], sc * dy)
                    plsc.addupdate_scatter(acc, [ival, cc(2)], sc * dz)
                    return c2

                lax.fori_loop(0, 8, rs_body, 0)
            return carry

        lax.fori_loop(0, NCH, chunk_body, 0)
        drain = Npad // 8
        for k in range(8):
            pltpu.sync_copy(acc.at[pl.ds(k * drain, drain)],
                            out_hbm.at[wid, pl.ds(k * drain, drain)])

    return edge_kernel


def _reduce_body(in_ref, out_ref):
    out_ref[...] = jnp.sum(in_ref[...], axis=0, keepdims=True)


def _reduce_partials(parts_flat):
    NW, F = parts_flat.shape
    return pl.pallas_call(
        _reduce_body,
        out_shape=jax.ShapeDtypeStruct((1, F), jnp.float32),
    )(parts_flat)


def kernel(atom_type, r_feat, p_feat, pos, pos_init, t, batch, edge_index,
           edge_type_r, edge_type_p, atom_emb_table, W_feat, W_node,
           edge_type_emb, W_dist, W_cat, W_msg, W_scalar):
    N = atom_type.shape[0]
    E = edge_index.shape[1]
    H = W_node.shape[1]

    ws = W_scalar[:, 0]
    Wc1, Wc2 = W_cat[:H], W_cat[H:]
    M0 = edge_type_emb * W_dist[0][None, :]
    M1 = edge_type_emb * W_dist[1][None, :]
    G = jnp.concatenate([M0 @ Wc1, M1 @ Wc1, M0 @ Wc2, M1 @ Wc2], axis=0)
    Wtot = W_node @ (W_msg * ws[None, :]) @ G.T
    Wt1, Wt2, wtt = Wtot[:H // 2], Wtot[H // 2:H], Wtot[H:]

    Npad = -(-N // (_NODE_BLK * 2)) * (_NODE_BLK * 2)
    pad_n = Npad - N
    i32 = jnp.int32
    at2 = jnp.pad(atom_type.astype(i32), (0, pad_n)).reshape(Npad, 1)
    bt2 = jnp.pad(batch.astype(i32), (0, pad_n)).reshape(Npad, 1)
    rf2 = jnp.pad(r_feat.astype(i32), ((0, pad_n), (0, 0)))
    pf2 = jnp.pad(p_feat.astype(i32), ((0, pad_n), (0, 0)))
    pos4 = jnp.pad(pos, ((0, pad_n), (0, 1)))
    pi4 = jnp.pad(pos_init, ((0, pad_n), (0, 1)))

    packI, packJ = _node_tables(at2, rf2, pf2, bt2, pos4, pi4,
                                atom_emb_table, W_feat, t.reshape(-1, 1),
                                Wt1, Wt2, wtt, Npad)

    chunk = _NC * _NS * _CH_ROWS * 128
    NCH = -(-E // chunk)
    Epad = NCH * chunk
    ER = Epad // 128
    pad_e = Epad - E
    ei = jnp.pad(edge_index[0].astype(i32), (0, pad_e)).reshape(ER, 128)
    ej = jnp.pad(edge_index[1].astype(i32), (0, pad_e)).reshape(ER, 128)
    er = jnp.pad(edge_type_r.astype(i32), (0, pad_e)).reshape(ER, 128)
    ep = jnp.pad(edge_type_p.astype(i32), (0, pad_e)).reshape(ER, 128)

    ek = _make_edge_kernel(Npad, ER, NCH)
    parts = ek(ei, ej, er, ep, packI, packJ)
    eps = _reduce_partials(parts.reshape(_NC * _NS, Npad * 4))
    return eps.reshape(Npad, 4)[:N, :3]


# speedup vs baseline: 17.0984x; 1.5747x over previous
import functools

import jax
import jax.numpy as jnp
from jax import lax
from jax.experimental import pallas as pl
from jax.experimental.pallas import tpu as pltpu
from jax.experimental.pallas import tpu_sc as plsc

_NC = 2
_NS = 16
_LANES = 16
_CH_ROWS = 8
_NODE_BLK = 1024


def _nrsqrt(x):
    k = lax.bitcast_convert_type(x, jnp.int32)
    y = lax.bitcast_convert_type(jnp.int32(0x5F3759DF) - (k >> 1), jnp.float32)
    for _ in range(3):
        y = y * (1.5 - 0.5 * x * y * y)
    return y


def _node_body(at_ref, rf_ref, pf_ref, bt_ref, pos_ref, pi_ref, tab_ref,
               wf0_ref, wf1_ref, wf2_ref, wf3_ref, t_ref, wt1_ref, wt2_ref,
               wtt_ref, packI_ref, packJ_ref):
    f32 = jnp.float32
    B = at_ref.shape[0]
    NT = tab_ref.shape[0]
    G = t_ref.shape[0]
    aoh = (at_ref[...] == lax.broadcasted_iota(jnp.int32, (B, NT), 1)).astype(f32)
    ae = jnp.dot(aoh, tab_ref[...], preferred_element_type=f32)
    wfs = (wf0_ref, wf1_ref, wf2_ref, wf3_ref)

    def featsum(ref):
        acc = None
        for f in range(4):
            oh = (ref[:, f:f + 1] == lax.broadcasted_iota(jnp.int32, (B, 10), 1)).astype(f32)
            v = jnp.dot(oh, wfs[f][...], preferred_element_type=f32)
            acc = v if acc is None else acc + v
        return acc

    rW = featsum(rf_ref)
    pW = featsum(pf_ref)
    boh = (bt_ref[...] == lax.broadcasted_iota(jnp.int32, (B, G), 1)).astype(f32)
    tn = jnp.dot(boh, t_ref[...], preferred_element_type=f32)
    abcd = (jnp.dot(ae * rW, wt1_ref[...], preferred_element_type=f32)
            + jnp.dot(ae * pW, wt2_ref[...], preferred_element_type=f32)
            + tn * wtt_ref[...])
    packI_ref[...] = jnp.concatenate([pos_ref[...], pi_ref[...]], axis=1)
    packJ_ref[...] = jnp.concatenate(
        [pos_ref[...], pi_ref[...], abcd, jnp.zeros((B, 4), f32)], axis=1)


def _node_tables(atom_type, r_feat, p_feat, batch, pos4, pi4, table, W_feat,
                 t, Wt1, Wt2, wtt, Npad):
    nblk = Npad // _NODE_BLK
    row = lambda b: (b, 0)
    rep = lambda b: (0, 0)
    blk = lambda c: pl.BlockSpec((_NODE_BLK, c), row)
    full = lambda a: pl.BlockSpec(a.shape, rep)
    wf = [W_feat[10 * f:10 * (f + 1)] for f in range(4)]
    return pl.pallas_call(
        _node_body,
        grid=(nblk,),
        in_specs=[blk(1), blk(4), blk(4), blk(1), blk(4), blk(4),
                  full(table), full(wf[0]), full(wf[1]), full(wf[2]),
                  full(wf[3]), full(t), full(Wt1), full(Wt2), full(wtt)],
        out_specs=[blk(8), blk(32)],
        out_shape=[jax.ShapeDtypeStruct((Npad, 8), jnp.float32),
                   jax.ShapeDtypeStruct((Npad, 32), jnp.float32)],
    )(atom_type, r_feat, p_feat, batch, pos4, pi4, table, wf[0], wf[1],
      wf[2], wf[3], t, Wt1, Wt2, wtt)


_SC_MESH = dict(core_axis_name="c", subcore_axis_name="s",
                num_cores=_NC, num_subcores=_NS)
_SC_PARAMS = dict(
    compiler_params=pltpu.CompilerParams(needs_layout_passes=False,
                                         use_tc_tiling_on_sc=False))


def _make_edge_kernel(Npad, Epad, NCH):
    NW = _NC * _NS
    CH = _CH_ROWS * 128
    edges_w = NCH * CH
    A4 = Npad * 4

    @functools.partial(
        pl.kernel,
        out_type=jax.ShapeDtypeStruct((NW, A4), jnp.float32),
        mesh=plsc.VectorSubcoreMesh(**_SC_MESH),
        **_SC_PARAMS,
        scratch_types=[
            pltpu.VMEM((CH,), jnp.int32),
            pltpu.VMEM((CH,), jnp.int32),
            pltpu.VMEM((CH,), jnp.int32),
            pltpu.VMEM((CH,), jnp.int32),
            *[pltpu.VMEM((128, 8), jnp.float32) for _ in range(_CH_ROWS)],
            *[pltpu.VMEM((128, 32), jnp.float32) for _ in range(_CH_ROWS)],
            pltpu.VMEM((A4,), jnp.float32),
            pltpu.SemaphoreType.DMA,
            pltpu.SemaphoreType.DMA,
        ],
    )
    def edge_kernel(ei_hbm, ej_hbm, etr_hbm, etp_hbm, packI_hbm, packJ_hbm,
                    out_hbm, iv, jv, erv, epv, *rest):
        R = _CH_ROWS
        bufI = rest[0:R]
        bufJ = rest[R:2 * R]
        acc = rest[2 * R]
        semi = rest[2 * R + 1]
        semg = rest[2 * R + 2]
        cid = lax.axis_index("c")
        sid = lax.axis_index("s")
        wid = cid * _NS + sid
        lanes = lax.iota(jnp.int32, _LANES)
        cc = lambda c: jnp.full((_LANES,), c, jnp.int32)

        def zero_body(k, c):
            plsc.store_scatter(acc, [k * _LANES + lanes],
                               jnp.zeros((_LANES,), jnp.float32))
            return c
        lax.fori_loop(0, A4 // _LANES, zero_body, 0)

        base = wid * edges_w

        def chunk_body(ch, carry):
            e0 = base + ch * CH
            cps = [pltpu.async_copy(ei_hbm.at[pl.ds(e0, CH)], iv, semi),
                   pltpu.async_copy(ej_hbm.at[pl.ds(e0, CH)], jv, semi),
                   pltpu.async_copy(etr_hbm.at[pl.ds(e0, CH)], erv, semi),
                   pltpu.async_copy(etp_hbm.at[pl.ds(e0, CH)], epv, semi)]
            for cp in cps:
                cp.wait()
            cps = []
            for r in range(R):
                cps.append(pltpu.async_copy(
                    packI_hbm.at[iv.at[pl.ds(r * 128, 128)]], bufI[r], semg))
                cps.append(pltpu.async_copy(
                    packJ_hbm.at[jv.at[pl.ds(r * 128, 128)]], bufJ[r], semg))
            for cp in cps:
                cp.wait()

            for r in range(R):
                def rs_body(s, c2, r=r):
                    li = s * _LANES + lanes
                    ei_l = r * 128 + li

                    def colI(c):
                        return plsc.load_gather(bufI[r], [li, cc(c)])

                    def colJ(cv):
                        return plsc.load_gather(bufJ[r], [li, cv])

                    pix, piy, piz = colI(0), colI(1), colI(2)
                    qix, qiy, qiz = colI(4), colI(5), colI(6)
                    pjx, pjy, pjz = colJ(cc(0)), colJ(cc(1)), colJ(cc(2))
                    qjx, qjy, qjz = colJ(cc(4)), colJ(cc(5)), colJ(cc(6))
                    etr = plsc.load_gather(erv, [ei_l])
                    etp = plsc.load_gather(epv, [ei_l])
                    Av = colJ(8 + etr)
                    Bv = colJ(13 + etr)
                    Cv = colJ(18 + etp)
                    Dv = colJ(23 + etp)
                    ival = plsc.load_gather(iv, [ei_l])
                    dx = pix - pjx
                    dy = piy - pjy
                    dz = piz - pjz
                    dd = dx * dx + dy * dy + dz * dz + 1e-8
                    y = _nrsqrt(dd)
                    tx = qix - qjx
                    ty = qiy - qjy
                    tz = qiz - qjz
                    ddT = tx * tx + ty * ty + tz * tz + 1e-8
                    yT = _nrsqrt(ddT)
                    coef = (dd * y) * (Av + Cv) + (ddT * yT) * (Bv + Dv)
                    sc = coef * y
                    i4 = ival * 4
                    plsc.addupdate_scatter(acc, [i4], sc * dx)
                    plsc.addupdate_scatter(acc, [i4 + 1], sc * dy)
                    plsc.addupdate_scatter(acc, [i4 + 2], sc * dz)
                    return c2

                lax.fori_loop(0, 8, rs_body, 0)
            return carry

        lax.fori_loop(0, NCH, chunk_body, 0)
        drain = A4 // 8
        for k in range(8):
            pltpu.sync_copy(acc.at[pl.ds(k * drain, drain)],
                            out_hbm.at[wid, pl.ds(k * drain, drain)])

    return edge_kernel


def _make_reduce_kernel(Npad):
    NW = _NC * _NS
    A4 = Npad * 4
    SL = A4 // NW

    @functools.partial(
        pl.kernel,
        out_type=jax.ShapeDtypeStruct((A4,), jnp.float32),
        mesh=plsc.VectorSubcoreMesh(**_SC_MESH),
        **_SC_PARAMS,
        scratch_types=[
            pltpu.VMEM((NW, SL), jnp.float32),
            pltpu.VMEM((SL,), jnp.float32),
            pltpu.SemaphoreType.DMA,
        ],
    )
    def reduce_kernel(parts_hbm, out_hbm, tmp, outv, sem):
        cid = lax.axis_index("c")
        sid = lax.axis_index("s")
        wid = cid * _NS + sid
        o0 = wid * SL
        lanes = lax.iota(jnp.int32, _LANES)
        cps = [pltpu.async_copy(parts_hbm.at[k, pl.ds(o0, SL)],
                                tmp.at[k], sem) for k in range(NW)]
        for cp in cps:
            cp.wait()

        def body(g, c):
            li = g * _LANES + lanes
            s = plsc.load_gather(tmp, [jnp.full((_LANES,), 0, jnp.int32), li])
            for k in range(1, NW):
                s = s + plsc.load_gather(
                    tmp, [jnp.full((_LANES,), k, jnp.int32), li])
            plsc.store_scatter(outv, [li], s)
            return c
        lax.fori_loop(0, SL // _LANES, body, 0)
        pltpu.sync_copy(outv, out_hbm.at[pl.ds(o0, SL)])

    return reduce_kernel


def kernel(atom_type, r_feat, p_feat, pos, pos_init, t, batch, edge_index,
           edge_type_r, edge_type_p, atom_emb_table, W_feat, W_node,
           edge_type_emb, W_dist, W_cat, W_msg, W_scalar):
    N = atom_type.shape[0]
    E = edge_index.shape[1]
    H = W_node.shape[1]

    ws = W_scalar[:, 0]
    Wc1, Wc2 = W_cat[:H], W_cat[H:]
    M0 = edge_type_emb * W_dist[0][None, :]
    M1 = edge_type_emb * W_dist[1][None, :]
    G = jnp.concatenate([M0 @ Wc1, M1 @ Wc1, M0 @ Wc2, M1 @ Wc2], axis=0)
    Wtot = W_node @ (W_msg * ws[None, :]) @ G.T
    Wt1, Wt2, wtt = Wtot[:H // 2], Wtot[H // 2:H], Wtot[H:]

    Npad = -(-N // (_NODE_BLK * 2)) * (_NODE_BLK * 2)
    pad_n = Npad - N
    i32 = jnp.int32
    at2 = jnp.pad(atom_type.astype(i32), (0, pad_n)).reshape(Npad, 1)
    bt2 = jnp.pad(batch.astype(i32), (0, pad_n)).reshape(Npad, 1)
    rf2 = jnp.pad(r_feat.astype(i32), ((0, pad_n), (0, 0)))
    pf2 = jnp.pad(p_feat.astype(i32), ((0, pad_n), (0, 0)))
    pos4 = jnp.pad(pos, ((0, pad_n), (0, 1)))
    pi4 = jnp.pad(pos_init, ((0, pad_n), (0, 1)))

    packI, packJ = _node_tables(at2, rf2, pf2, bt2, pos4, pi4,
                                atom_emb_table, W_feat, t.reshape(-1, 1),
                                Wt1, Wt2, wtt, Npad)

    chunk = _NC * _NS * _CH_ROWS * 128
    NCH = -(-E // chunk)
    Epad = NCH * chunk
    pad_e = Epad - E
    ei = jnp.pad(edge_index[0].astype(i32), (0, pad_e))
    ej = jnp.pad(edge_index[1].astype(i32), (0, pad_e))
    er = jnp.pad(edge_type_r.astype(i32), (0, pad_e))
    ep = jnp.pad(edge_type_p.astype(i32), (0, pad_e))

    ek = _make_edge_kernel(Npad, Epad, NCH)
    parts = ek(ei, ej, er, ep, packI, packJ)
    eps = _make_reduce_kernel(Npad)(parts)
    return eps.reshape(Npad, 4)[:N, :3]


# speedup vs baseline: 21.3048x; 1.2460x over previous
import functools

import jax
import jax.numpy as jnp
from jax import lax
from jax.experimental import pallas as pl
from jax.experimental.pallas import tpu as pltpu
from jax.experimental.pallas import tpu_sc as plsc

_NC = 2
_NS = 16
_LANES = 16
_CH_ROWS = 4
_NODE_BLK = 1024


def _nrsqrt(x):
    k = lax.bitcast_convert_type(x, jnp.int32)
    y = lax.bitcast_convert_type(jnp.int32(0x5F3759DF) - (k >> 1), jnp.float32)
    for _ in range(3):
        y = y * (1.5 - 0.5 * x * y * y)
    return y


def _node_body(at_ref, rf_ref, pf_ref, bt_ref, pos_ref, pi_ref, tab_ref,
               wf0_ref, wf1_ref, wf2_ref, wf3_ref, t_ref, wt1_ref, wt2_ref,
               wtt_ref, packI_ref, packJ_ref):
    f32 = jnp.float32
    B = at_ref.shape[0]
    NT = tab_ref.shape[0]
    G = t_ref.shape[0]
    aoh = (at_ref[...] == lax.broadcasted_iota(jnp.int32, (B, NT), 1)).astype(f32)
    ae = jnp.dot(aoh, tab_ref[...], preferred_element_type=f32)
    wfs = (wf0_ref, wf1_ref, wf2_ref, wf3_ref)

    def featsum(ref):
        acc = None
        for f in range(4):
            oh = (ref[:, f:f + 1] == lax.broadcasted_iota(jnp.int32, (B, 10), 1)).astype(f32)
            v = jnp.dot(oh, wfs[f][...], preferred_element_type=f32)
            acc = v if acc is None else acc + v
        return acc

    rW = featsum(rf_ref)
    pW = featsum(pf_ref)
    boh = (bt_ref[...] == lax.broadcasted_iota(jnp.int32, (B, G), 1)).astype(f32)
    tn = jnp.dot(boh, t_ref[...], preferred_element_type=f32)
    abcd = (jnp.dot(ae * rW, wt1_ref[...], preferred_element_type=f32)
            + jnp.dot(ae * pW, wt2_ref[...], preferred_element_type=f32)
            + tn * wtt_ref[...])
    packI_ref[...] = jnp.concatenate([pos_ref[...], pi_ref[...]], axis=1)
    packJ_ref[...] = jnp.concatenate(
        [pos_ref[...], pi_ref[...], abcd, jnp.zeros((B, 4), f32)], axis=1)


def _node_tables(atom_type, r_feat, p_feat, batch, pos4, pi4, table, W_feat,
                 t, Wt1, Wt2, wtt, Npad):
    nblk = Npad // _NODE_BLK
    row = lambda b: (b, 0)
    rep = lambda b: (0, 0)
    blk = lambda c: pl.BlockSpec((_NODE_BLK, c), row)
    full = lambda a: pl.BlockSpec(a.shape, rep)
    wf = [W_feat[10 * f:10 * (f + 1)] for f in range(4)]
    return pl.pallas_call(
        _node_body,
        grid=(nblk,),
        in_specs=[blk(1), blk(4), blk(4), blk(1), blk(4), blk(4),
                  full(table), full(wf[0]), full(wf[1]), full(wf[2]),
                  full(wf[3]), full(t), full(Wt1), full(Wt2), full(wtt)],
        out_specs=[blk(8), blk(32)],
        out_shape=[jax.ShapeDtypeStruct((Npad, 8), jnp.float32),
                   jax.ShapeDtypeStruct((Npad, 32), jnp.float32)],
    )(atom_type, r_feat, p_feat, batch, pos4, pi4, table, wf[0], wf[1],
      wf[2], wf[3], t, Wt1, Wt2, wtt)


_SC_MESH = dict(core_axis_name="c", subcore_axis_name="s",
                num_cores=_NC, num_subcores=_NS)
_SC_PARAMS = dict(
    compiler_params=pltpu.CompilerParams(needs_layout_passes=False,
                                         use_tc_tiling_on_sc=False))


def _make_edge_kernel(Npad, Epad, NCH):
    NW = _NC * _NS
    R = _CH_ROWS
    CH = R * 128
    edges_w = NCH * CH
    A4 = Npad * 4

    @functools.partial(
        pl.kernel,
        out_type=jax.ShapeDtypeStruct((NW, A4), jnp.float32),
        mesh=plsc.VectorSubcoreMesh(**_SC_MESH),
        **_SC_PARAMS,
        scratch_types=[
            *[pltpu.VMEM((CH,), jnp.int32) for _ in range(8)],
            *[pltpu.VMEM((128, 8), jnp.float32) for _ in range(2 * R)],
            *[pltpu.VMEM((128, 32), jnp.float32) for _ in range(2 * R)],
            pltpu.VMEM((A4,), jnp.float32),
            pltpu.SemaphoreType.DMA,
            pltpu.SemaphoreType.DMA,
            pltpu.SemaphoreType.DMA,
            pltpu.SemaphoreType.DMA,
        ],
    )
    def edge_kernel(ei_hbm, ej_hbm, etr_hbm, etp_hbm, packI_hbm, packJ_hbm,
                    out_hbm, *rest):
        idxb = [rest[0:4], rest[4:8]]
        bufI = [rest[8:8 + R], rest[8 + R:8 + 2 * R]]
        bufJ = [rest[8 + 2 * R:8 + 3 * R], rest[8 + 3 * R:8 + 4 * R]]
        acc = rest[8 + 4 * R]
        semi = [rest[8 + 4 * R + 1], rest[8 + 4 * R + 2]]
        semg = [rest[8 + 4 * R + 3], rest[8 + 4 * R + 4]]
        cid = lax.axis_index("c")
        sid = lax.axis_index("s")
        wid = cid * _NS + sid
        lanes = lax.iota(jnp.int32, _LANES)
        cc = lambda c: jnp.full((_LANES,), c, jnp.int32)

        def zero_body(k, c):
            plsc.store_scatter(acc, [k * _LANES + lanes],
                               jnp.zeros((_LANES,), jnp.float32))
            return c
        lax.fori_loop(0, A4 // _LANES, zero_body, 0)

        base = wid * edges_w
        ehbm = [ei_hbm, ej_hbm, etr_hbm, etp_hbm]

        def fire_idx(ch, p):
            e0 = base + ch * CH
            for a in range(4):
                pltpu.async_copy(ehbm[a].at[pl.ds(e0, CH)], idxb[p][a],
                                 semi[p])

        def wait_idx(p):
            for a in range(4):
                pltpu.make_async_copy(ehbm[a].at[pl.ds(0, CH)], idxb[p][a],
                                      semi[p]).wait()

        def fire_gather(p):
            iv, jv = idxb[p][0], idxb[p][1]
            for r in range(R):
                pltpu.async_copy(packI_hbm.at[iv.at[pl.ds(r * 128, 128)]],
                                 bufI[p][r], semg[p])
                pltpu.async_copy(packJ_hbm.at[jv.at[pl.ds(r * 128, 128)]],
                                 bufJ[p][r], semg[p])

        def wait_gather(p):
            iv, jv = idxb[p][0], idxb[p][1]
            for r in range(R):
                pltpu.make_async_copy(
                    packI_hbm.at[iv.at[pl.ds(r * 128, 128)]],
                    bufI[p][r], semg[p]).wait()
                pltpu.make_async_copy(
                    packJ_hbm.at[jv.at[pl.ds(r * 128, 128)]],
                    bufJ[p][r], semg[p]).wait()

        def compute(p):
            iv, erv, epv = idxb[p][0], idxb[p][2], idxb[p][3]
            for r in range(R):
                def rs_body(s, c2, r=r):
                    li = s * _LANES + lanes
                    e_l = r * 128 + s * _LANES

                    def colI(c):
                        return plsc.load_gather(bufI[p][r], [li, cc(c)])

                    def colJ(cv):
                        return plsc.load_gather(bufJ[p][r], [li, cv])

                    pix, piy, piz = colI(0), colI(1), colI(2)
                    qix, qiy, qiz = colI(4), colI(5), colI(6)
                    pjx, pjy, pjz = colJ(cc(0)), colJ(cc(1)), colJ(cc(2))
                    qjx, qjy, qjz = colJ(cc(4)), colJ(cc(5)), colJ(cc(6))
                    etr = erv[pl.ds(e_l, _LANES)]
                    etp = epv[pl.ds(e_l, _LANES)]
                    Av = colJ(8 + etr)
                    Bv = colJ(13 + etr)
                    Cv = colJ(18 + etp)
                    Dv = colJ(23 + etp)
                    ival = iv[pl.ds(e_l, _LANES)]
                    dx = pix - pjx
                    dy = piy - pjy
                    dz = piz - pjz
                    dd = dx * dx + dy * dy + dz * dz + 1e-8
                    y = _nrsqrt(dd)
                    tx = qix - qjx
                    ty = qiy - qjy
                    tz = qiz - qjz
                    ddT = tx * tx + ty * ty + tz * tz + 1e-8
                    yT = _nrsqrt(ddT)
                    coef = (dd * y) * (Av + Cv) + (ddT * yT) * (Bv + Dv)
                    sc = coef * y
                    i4 = ival * 4
                    plsc.addupdate_scatter(acc, [i4], sc * dx)
                    plsc.addupdate_scatter(acc, [i4 + 1], sc * dy)
                    plsc.addupdate_scatter(acc, [i4 + 2], sc * dz)
                    return c2

                lax.fori_loop(0, 8, rs_body, 0)

        fire_idx(0, 0)
        fire_idx(1, 1)
        wait_idx(0)
        fire_gather(0)

        def pair_body(k, carry):
            a = 2 * k
            wait_idx(1)
            fire_gather(1)
            wait_gather(0)
            compute(0)
            fire_idx(a + 2, 0)
            wait_gather(1)
            wait_idx(0)
            fire_gather(0)
            compute(1)
            fire_idx(a + 3, 1)
            return carry

        lax.fori_loop(0, NCH // 2, pair_body, 0)
        wait_gather(0)
        wait_idx(1)
        drain = A4 // 8
        for k in range(8):
            pltpu.sync_copy(acc.at[pl.ds(k * drain, drain)],
                            out_hbm.at[wid, pl.ds(k * drain, drain)])

    return edge_kernel


def _make_reduce_kernel(Npad):
    NW = _NC * _NS
    A4 = Npad * 4
    SL = A4 // NW

    @functools.partial(
        pl.kernel,
        out_type=jax.ShapeDtypeStruct((A4,), jnp.float32),
        mesh=plsc.VectorSubcoreMesh(**_SC_MESH),
        **_SC_PARAMS,
        scratch_types=[
            pltpu.VMEM((NW, SL), jnp.float32),
            pltpu.VMEM((SL,), jnp.float32),
            pltpu.SemaphoreType.DMA,
        ],
    )
    def reduce_kernel(parts_hbm, out_hbm, tmp, outv, sem):
        cid = lax.axis_index("c")
        sid = lax.axis_index("s")
        wid = cid * _NS + sid
        o0 = wid * SL
        lanes = lax.iota(jnp.int32, _LANES)
        cps = [pltpu.async_copy(parts_hbm.at[k, pl.ds(o0, SL)],
                                tmp.at[k], sem) for k in range(NW)]
        for cp in cps:
            cp.wait()

        def body(g, c):
            li = g * _LANES + lanes
            s = plsc.load_gather(tmp, [jnp.full((_LANES,), 0, jnp.int32), li])
            for k in range(1, NW):
                s = s + plsc.load_gather(
                    tmp, [jnp.full((_LANES,), k, jnp.int32), li])
            plsc.store_scatter(outv, [li], s)
            return c
        lax.fori_loop(0, SL // _LANES, body, 0)
        pltpu.sync_copy(outv, out_hbm.at[pl.ds(o0, SL)])

    return reduce_kernel


def kernel(atom_type, r_feat, p_feat, pos, pos_init, t, batch, edge_index,
           edge_type_r, edge_type_p, atom_emb_table, W_feat, W_node,
           edge_type_emb, W_dist, W_cat, W_msg, W_scalar):
    N = atom_type.shape[0]
    E = edge_index.shape[1]
    H = W_node.shape[1]

    ws = W_scalar[:, 0]
    Wc1, Wc2 = W_cat[:H], W_cat[H:]
    M0 = edge_type_emb * W_dist[0][None, :]
    M1 = edge_type_emb * W_dist[1][None, :]
    G = jnp.concatenate([M0 @ Wc1, M1 @ Wc1, M0 @ Wc2, M1 @ Wc2], axis=0)
    Wtot = W_node @ (W_msg * ws[None, :]) @ G.T
    Wt1, Wt2, wtt = Wtot[:H // 2], Wtot[H // 2:H], Wtot[H:]

    Npad = -(-N // (_NODE_BLK * 2)) * (_NODE_BLK * 2)
    pad_n = Npad - N
    i32 = jnp.int32
    at2 = jnp.pad(atom_type.astype(i32), (0, pad_n)).reshape(Npad, 1)
    bt2 = jnp.pad(batch.astype(i32), (0, pad_n)).reshape(Npad, 1)
    rf2 = jnp.pad(r_feat.astype(i32), ((0, pad_n), (0, 0)))
    pf2 = jnp.pad(p_feat.astype(i32), ((0, pad_n), (0, 0)))
    pos4 = jnp.pad(pos, ((0, pad_n), (0, 1)))
    pi4 = jnp.pad(pos_init, ((0, pad_n), (0, 1)))

    packI, packJ = _node_tables(at2, rf2, pf2, bt2, pos4, pi4,
                                atom_emb_table, W_feat, t.reshape(-1, 1),
                                Wt1, Wt2, wtt, Npad)

    CH = _CH_ROWS * 128
    chunk = _NC * _NS * CH
    NCH = -(-E // chunk)
    NCH = NCH + (NCH % 2)
    Epad = NCH * chunk
    pad_e = Epad - E + 2 * CH
    ei = jnp.pad(edge_index[0].astype(i32), (0, pad_e))
    ej = jnp.pad(edge_index[1].astype(i32), (0, pad_e))
    er = jnp.pad(edge_type_r.astype(i32), (0, pad_e))
    ep = jnp.pad(edge_type_p.astype(i32), (0, pad_e))

    ek = _make_edge_kernel(Npad, Epad, NCH)
    parts = ek(ei, ej, er, ep, packI, packJ)
    eps = _make_reduce_kernel(Npad)(parts)
    return eps.reshape(Npad, 4)[:N, :3]
